# Initial kernel scaffold; baseline (speedup 1.0000x reference)
import jax
import jax.numpy as jnp
from jax.experimental import pallas as pl


def kernel(input_context, input_query, table):
    raise NotImplementedError("write your pallas kernel here")


# speedup vs baseline: 7.1850x; 7.1850x over previous
import functools

import jax
import jax.numpy as jnp
from jax import lax
from jax.experimental import pallas as pl
from jax.experimental.pallas import tpu as pltpu
from jax.experimental.pallas import tpu_sc as plsc

_NW = 32
_CHUNK = 128
_NBUF = 4


def _sc_double_gather(idx_ctx, idx_qry, table, g_ctx, g_qry):
    dim = table.shape[1]
    mesh = plsc.VectorSubcoreMesh(core_axis_name="c", subcore_axis_name="s")

    @functools.partial(
        pl.kernel,
        mesh=mesh,
        out_type=[
            jax.ShapeDtypeStruct((_NW, g_ctx, _CHUNK, dim), jnp.float32),
            jax.ShapeDtypeStruct((_NW, g_qry, _CHUNK, dim), jnp.float32),
        ],
        scratch_types=[
            pltpu.VMEM((g_ctx, _CHUNK), jnp.int32),
            pltpu.VMEM((g_qry, _CHUNK), jnp.int32),
            pltpu.VMEM((_NBUF, _CHUNK, dim), jnp.float32),
        ]
        + [pltpu.SemaphoreType.DMA] * (2 * _NBUF),
    )
    def run(ctx_hbm, qry_hbm, table_hbm, ctx_out, qry_out,
            ctx_idx_v, qry_idx_v, rows_v, *sems):
        gsems = sems[:_NBUF]
        ssems = sems[_NBUF:]
        wid = lax.axis_index("s") * 2 + lax.axis_index("c")

        pltpu.sync_copy(ctx_hbm.at[wid], ctx_idx_v)
        pltpu.sync_copy(qry_hbm.at[wid], qry_idx_v)

        def stream(idx_v, out_hbm, n_chunks):
            def g_start(j, b):
                pltpu.async_copy(
                    table_hbm.at[idx_v.at[j]], rows_v.at[b], gsems[b])

            def g_wait(j, b):
                pltpu.make_async_copy(
                    table_hbm.at[idx_v.at[j]], rows_v.at[b], gsems[b]).wait()

            def s_start(j, b):
                pltpu.async_copy(rows_v.at[b], out_hbm.at[wid, j], ssems[b])

            def s_wait(j, b):
                pltpu.make_async_copy(
                    rows_v.at[b], out_hbm.at[wid, j], ssems[b]).wait()

            for b in range(_NBUF):
                g_start(b, b)

            def body(i, carry):
                base = i * _NBUF
                for b in range(_NBUF):
                    j = base + b
                    g_wait(j, b)
                    s_start(j, b)
                    s_wait(j, b)
                    nj = j + _NBUF

                    @pl.when(nj < n_chunks)
                    def _():
                        g_start(nj, b)
                return carry

            lax.fori_loop(0, n_chunks // _NBUF, body, 0)

        stream(ctx_idx_v, ctx_out, g_ctx)
        stream(qry_idx_v, qry_out, g_qry)

    return run(idx_ctx, idx_qry, table)


def kernel(input_context, input_query, table):
    b, ctx_len = input_context.shape
    _, qry_len = input_query.shape
    dim = table.shape[1]

    n_ctx = b * ctx_len
    n_qry = b * qry_len
    g_ctx = n_ctx // (_NW * _CHUNK)
    g_qry = n_qry // (_NW * _CHUNK)

    idx_ctx = input_context.reshape(_NW, g_ctx, _CHUNK)
    idx_qry = input_query.reshape(_NW, g_qry, _CHUNK)

    ctx_o, qry_o = _sc_double_gather(idx_ctx, idx_qry, table, g_ctx, g_qry)
    return (ctx_o.reshape(b, ctx_len, dim), qry_o.reshape(b, qry_len, dim))
